# baseline (speedup 1.0000x reference)
import jax
import jax.numpy as jnp
from jax import lax
from jax.experimental import pallas as pl
from jax.experimental.pallas import tpu as pltpu
from jax.experimental.pallas import tpu_sc as plsc

B = 4
S = 2048
D = 768
NC = 2
NS = 16
NW = NC * NS
P = S // NW
C = 16
G = P // C
R = B * C
LANES = 16
NCOL = D // LANES


def _body(tok_hbm, emb_hbm, pos_hbm, out_hbm,
          idx0, idx1, pos0, pos1, buf0, buf1,
          ps0, ps1, gs0, gs1, os0, os1):
    wid = lax.axis_index("s") * NC + lax.axis_index("c")
    p0 = wid * P

    idxs = (idx0, idx1)
    poss = (pos0, pos1)
    bufs = (buf0, buf1)
    psems = (ps0, ps1)
    gsems = (gs0, gs1)
    osems = (os0, os1)

    def start_group(g):
        s = g % 2
        for b in range(B):
            pltpu.sync_copy(tok_hbm.at[b, pl.ds(p0 + g * C, C)],
                            idxs[s].at[pl.ds(b * C, C)])
        pcp = pltpu.async_copy(
            pos_hbm.at[pl.ds(p0 + g * C, C)], poss[s], psems[s])
        gcp = pltpu.async_copy(emb_hbm.at[idxs[s]], bufs[s], gsems[s])
        return pcp, gcp

    grp = [start_group(0), start_group(1), None, None]
    outs = [None] * G

    for g in range(G):
        s = g % 2
        pcp, gcp = grp[g]
        pcp.wait()
        gcp.wait()
        pos_s = poss[s]
        buf = bufs[s]

        @plsc.parallel_loop(0, C, 1, unroll=2)
        def _(r, pos_s=pos_s, buf=buf):
            for c in range(NCOL):
                sl = pl.ds(c * LANES, LANES)
                pv = pos_s[r, sl]
                for q in range(B):
                    buf[q * C + r, sl] = buf[q * C + r, sl] + pv

        outs[g] = [
            pltpu.async_copy(buf.at[pl.ds(q * C, C)],
                             out_hbm.at[q, pl.ds(p0 + g * C, C)], osems[s])
            for q in range(B)
        ]
        if g + 2 < G:
            for cp in outs[g]:
                cp.wait()
            grp[g + 2] = start_group(g + 2)

    for g in (G - 2, G - 1):
        for cp in outs[g]:
            cp.wait()


@jax.jit
def _run(tokens, emb, pos_emb):
    mesh = plsc.VectorSubcoreMesh(core_axis_name="c", subcore_axis_name="s")
    f = pl.kernel(
        _body,
        out_type=jax.ShapeDtypeStruct((B, S, D), jnp.float32),
        mesh=mesh,
        scratch_types=[
            pltpu.VMEM((R,), jnp.int32),
            pltpu.VMEM((R,), jnp.int32),
            pltpu.VMEM((C, D), jnp.float32),
            pltpu.VMEM((C, D), jnp.float32),
            pltpu.VMEM((R, D), jnp.float32),
            pltpu.VMEM((R, D), jnp.float32),
            pltpu.SemaphoreType.DMA,
            pltpu.SemaphoreType.DMA,
            pltpu.SemaphoreType.DMA,
            pltpu.SemaphoreType.DMA,
            pltpu.SemaphoreType.DMA,
            pltpu.SemaphoreType.DMA,
        ],
    )
    return f(tokens, emb, pos_emb)


def kernel(tokens, emb, pos_emb):
    return _run(tokens.astype(jnp.int32), emb, pos_emb)


# speedup vs baseline: 1.8242x; 1.8242x over previous
import jax
import jax.numpy as jnp
from jax import lax
from jax.experimental import pallas as pl
from jax.experimental.pallas import tpu as pltpu
from jax.experimental.pallas import tpu_sc as plsc

B = 4
S = 2048
D = 768
NC = 2
NS = 16
NW = NC * NS
P = S // NW
C = 16
G = P // C
LANES = 16
NCOL = D // LANES


def _body(tok_hbm, emb_hbm, pos_hbm, out_hbm,
          idx_all, pos0, pos1,
          b00, b01, b02, b03, b10, b11, b12, b13,
          ps0, ps1, gs0, gs1, os0, os1):
    wid = lax.axis_index("s") * NC + lax.axis_index("c")
    p0 = wid * P

    poss = (pos0, pos1)
    bufsets = ((b00, b01, b02, b03), (b10, b11, b12, b13))
    psems = (ps0, ps1)
    gsems = (gs0, gs1)
    osems = (os0, os1)

    for b in range(B):
        pltpu.sync_copy(tok_hbm.at[b, pl.ds(p0, P)], idx_all.at[b])

    def start_pos(g):
        s = g % 2
        return pltpu.async_copy(
            pos_hbm.at[pl.ds(p0 + g * C, C)], poss[s], psems[s])

    def start_gather(g, b):
        s = g % 2
        return pltpu.async_copy(
            emb_hbm.at[idx_all.at[b, pl.ds(g * C, C)]],
            bufsets[s][b], gsems[s])

    def start_group(g):
        return start_pos(g), [start_gather(g, b) for b in range(B)]

    grp = [start_group(0), start_group(1), None, None]
    outs = [None] * G

    for g in range(G):
        s = g % 2
        pcp, gcps = grp[g]
        pcp.wait()
        for q in range(B):
            gcps[q].wait()
        pos_s = poss[s]
        bset = bufsets[s]

        @plsc.parallel_loop(0, C, 1, unroll=2)
        def _(r, pos_s=pos_s, bset=bset):
            for c in range(NCOL):
                sl = pl.ds(c * LANES, LANES)
                pv = pos_s[r, sl]
                for q in range(B):
                    bset[q][r, sl] = bset[q][r, sl] + pv

        outs[g] = [
            pltpu.async_copy(
                bset[q], out_hbm.at[q, pl.ds(p0 + g * C, C)], osems[s])
            for q in range(B)
        ]
        if g + 2 < G:
            pcp2 = start_pos(g + 2)
            gcps2 = []
            for q in range(B):
                outs[g][q].wait()
                gcps2.append(start_gather(g + 2, q))
            grp[g + 2] = (pcp2, gcps2)

    for g in (G - 2, G - 1):
        for cp in outs[g]:
            cp.wait()


@jax.jit
def _run(tokens, emb, pos_emb):
    mesh = plsc.VectorSubcoreMesh(core_axis_name="c", subcore_axis_name="s")
    buf = pltpu.VMEM((C, D), jnp.float32)
    f = pl.kernel(
        _body,
        out_type=jax.ShapeDtypeStruct((B, S, D), jnp.float32),
        mesh=mesh,
        scratch_types=[
            pltpu.VMEM((B, P), jnp.int32),
            buf, buf,
            buf, buf, buf, buf,
            buf, buf, buf, buf,
            pltpu.SemaphoreType.DMA,
            pltpu.SemaphoreType.DMA,
            pltpu.SemaphoreType.DMA,
            pltpu.SemaphoreType.DMA,
            pltpu.SemaphoreType.DMA,
            pltpu.SemaphoreType.DMA,
        ],
    )
    return f(tokens, emb, pos_emb)


def kernel(tokens, emb, pos_emb):
    return _run(tokens.astype(jnp.int32), emb, pos_emb)


# speedup vs baseline: 1.8632x; 1.0214x over previous
import jax
import jax.numpy as jnp
from jax import lax
from jax.experimental import pallas as pl
from jax.experimental.pallas import tpu as pltpu
from jax.experimental.pallas import tpu_sc as plsc

B = 4
S = 2048
D = 768
NC = 2
NS = 16
NW = NC * NS
P = S // NW
C = 16
G = P // C
LANES = 16
NCOL = D // LANES


def _body(tok_hbm, emb_hbm, pos_hbm, out_hbm,
          idx_all, pos0, pos1,
          b00, b01, b02, b03, b10, b11, b12, b13,
          ps0, ps1, gs0, gs1, os0, os1):
    wid = lax.axis_index("s") * NC + lax.axis_index("c")
    p0 = wid * P

    poss = (pos0, pos1)
    bufsets = ((b00, b01, b02, b03), (b10, b11, b12, b13))
    psems = (ps0, ps1)
    gsems = (gs0, gs1)
    osems = (os0, os1)

    idx_cps = [
        pltpu.async_copy(tok_hbm.at[b, pl.ds(p0, P)], idx_all.at[b], os0)
        for b in range(B)
    ]
    for cp in idx_cps:
        cp.wait()

    def start_pos(g):
        s = g % 2
        return pltpu.async_copy(
            pos_hbm.at[pl.ds(p0 + g * C, C)], poss[s], psems[s])

    def start_gather(g, b):
        s = g % 2
        return pltpu.async_copy(
            emb_hbm.at[idx_all.at[b, pl.ds(g * C, C)]],
            bufsets[s][b], gsems[s])

    def start_group(g):
        return start_pos(g), [start_gather(g, b) for b in range(B)]

    grp = [start_group(0), start_group(1), None, None]
    outs = [None] * G

    for g in range(G):
        s = g % 2
        pcp, gcps = grp[g]
        pcp.wait()
        for q in range(B):
            gcps[q].wait()
        pos_s = poss[s]
        bset = bufsets[s]

        @plsc.parallel_loop(0, C, 1, unroll=2)
        def _(r, pos_s=pos_s, bset=bset):
            for c in range(NCOL):
                sl = pl.ds(c * LANES, LANES)
                pv = pos_s[r, sl]
                for q in range(B):
                    bset[q][r, sl] = bset[q][r, sl] + pv

        outs[g] = [
            pltpu.async_copy(
                bset[q], out_hbm.at[q, pl.ds(p0 + g * C, C)], osems[s])
            for q in range(B)
        ]
        if g + 2 < G:
            pcp2 = start_pos(g + 2)
            gcps2 = []
            for q in range(B):
                outs[g][q].wait()
                gcps2.append(start_gather(g + 2, q))
            grp[g + 2] = (pcp2, gcps2)

    for g in (G - 2, G - 1):
        for cp in outs[g]:
            cp.wait()


@jax.jit
def _run(tokens, emb, pos_emb):
    mesh = plsc.VectorSubcoreMesh(core_axis_name="c", subcore_axis_name="s")
    buf = pltpu.VMEM((C, D), jnp.float32)
    f = pl.kernel(
        _body,
        out_type=jax.ShapeDtypeStruct((B, S, D), jnp.float32),
        mesh=mesh,
        scratch_types=[
            pltpu.VMEM((B, P), jnp.int32),
            buf, buf,
            buf, buf, buf, buf,
            buf, buf, buf, buf,
            pltpu.SemaphoreType.DMA,
            pltpu.SemaphoreType.DMA,
            pltpu.SemaphoreType.DMA,
            pltpu.SemaphoreType.DMA,
            pltpu.SemaphoreType.DMA,
            pltpu.SemaphoreType.DMA,
        ],
    )
    return f(tokens, emb, pos_emb)


def kernel(tokens, emb, pos_emb):
    return _run(tokens.astype(jnp.int32), emb, pos_emb)


# speedup vs baseline: 1.8835x; 1.0109x over previous
import jax
import jax.numpy as jnp
from jax import lax
from jax.experimental import pallas as pl
from jax.experimental.pallas import tpu as pltpu
from jax.experimental.pallas import tpu_sc as plsc

B = 4
S = 2048
D = 768
NC = 2
NS = 16
NW = NC * NS
P = S // NW
C = 16
G = P // C
LANES = 16
NCOL = D // LANES


def _body(tok_hbm, emb_hbm, pos_hbm, out_hbm,
          idx_all, pos0, pos1,
          b00, b01, b02, b03, b10, b11, b12, b13,
          ps0, ps1, gs0, gs1, os0, os1):
    wid = lax.axis_index("s") * NC + lax.axis_index("c")
    p0 = wid * P

    poss = (pos0, pos1)
    bufsets = ((b00, b01, b02, b03), (b10, b11, b12, b13))
    psems = (ps0, ps1)
    gsems = (gs0, gs1)
    osems = (os0, os1)

    def start_pos(g):
        s = g % 2
        return pltpu.async_copy(
            pos_hbm.at[pl.ds(p0 + g * C, C)], poss[s], psems[s])

    def start_gather(g, b):
        s = g % 2
        return pltpu.async_copy(
            emb_hbm.at[idx_all.at[b, pl.ds(g * C, C)]],
            bufsets[s][b], gsems[s])

    pcp0 = start_pos(0)
    pcp1 = start_pos(1)
    idx_cps = [
        pltpu.async_copy(tok_hbm.at[b, pl.ds(p0, P)], idx_all.at[b], os0)
        for b in range(B)
    ]
    g0 = []
    for b in range(B):
        idx_cps[b].wait()
        g0.append(start_gather(0, b))
    g1 = [start_gather(1, b) for b in range(B)]

    grp = [(pcp0, g0), (pcp1, g1), None, None]
    outs = [None] * G

    for g in range(G):
        s = g % 2
        pcp, gcps = grp[g]
        pcp.wait()
        for q in range(B):
            gcps[q].wait()
        pos_s = poss[s]
        bset = bufsets[s]

        @plsc.parallel_loop(0, C, 1, unroll=2)
        def _(r, pos_s=pos_s, bset=bset):
            for c in range(NCOL):
                sl = pl.ds(c * LANES, LANES)
                pv = pos_s[r, sl]
                for q in range(B):
                    bset[q][r, sl] = bset[q][r, sl] + pv

        outs[g] = [
            pltpu.async_copy(
                bset[q], out_hbm.at[q, pl.ds(p0 + g * C, C)], osems[s])
            for q in range(B)
        ]
        if g + 2 < G:
            pcp2 = start_pos(g + 2)
            gcps2 = []
            for q in range(B):
                outs[g][q].wait()
                gcps2.append(start_gather(g + 2, q))
            grp[g + 2] = (pcp2, gcps2)

    for g in (G - 2, G - 1):
        for cp in outs[g]:
            cp.wait()


@jax.jit
def _run(tokens, emb, pos_emb):
    mesh = plsc.VectorSubcoreMesh(core_axis_name="c", subcore_axis_name="s")
    buf = pltpu.VMEM((C, D), jnp.float32)
    f = pl.kernel(
        _body,
        out_type=jax.ShapeDtypeStruct((B, S, D), jnp.float32),
        mesh=mesh,
        scratch_types=[
            pltpu.VMEM((B, P), jnp.int32),
            buf, buf,
            buf, buf, buf, buf,
            buf, buf, buf, buf,
            pltpu.SemaphoreType.DMA,
            pltpu.SemaphoreType.DMA,
            pltpu.SemaphoreType.DMA,
            pltpu.SemaphoreType.DMA,
            pltpu.SemaphoreType.DMA,
            pltpu.SemaphoreType.DMA,
        ],
    )
    return f(tokens, emb, pos_emb)


def kernel(tokens, emb, pos_emb):
    return _run(tokens.astype(jnp.int32), emb, pos_emb)
